# baseline (speedup 1.0000x reference)
import jax
import jax.numpy as jnp
from jax.experimental import pallas as pl


_BLK = 256


def _diag_matmul_kernel(a_ref, b_ref, out_ref):
    out_ref[...] = jax.lax.dot_general(
        a_ref[...], b_ref[...],
        dimension_numbers=(((2,), (1,)), ((0,), (0,))),
        preferred_element_type=jnp.float32,
    )


def kernel(a, b):
    B, H, S, _ = a.shape
    D = b.shape[-1]
    a3 = a.reshape(B * H, S, S)
    b3 = b.reshape(B * H, S, D)
    n_blocks = S // _BLK

    out = pl.pallas_call(
        _diag_matmul_kernel,
        grid=(B * H, n_blocks),
        in_specs=[
            pl.BlockSpec((1, _BLK, _BLK), lambda h, i: (h, i, i)),
            pl.BlockSpec((1, _BLK, D), lambda h, i: (h, i, 0)),
        ],
        out_specs=pl.BlockSpec((1, _BLK, D), lambda h, i: (h, i, 0)),
        out_shape=jax.ShapeDtypeStruct((B * H, S, D), jnp.float32),
    )(a3, b3)

    return out.reshape(B, H, S, D)


# speedup vs baseline: 1.9030x; 1.9030x over previous
import jax
import jax.numpy as jnp
from jax.experimental import pallas as pl


_BLK = 256


def _diag_matmul_kernel(a_ref, b_ref, out_ref):
    out_ref[...] = jax.lax.dot_general(
        a_ref[...], b_ref[...],
        dimension_numbers=(((2,), (1,)), ((0,), (0,))),
        preferred_element_type=jnp.float32,
    )


def kernel(a, b):
    B, H, S, _ = a.shape
    D = b.shape[-1]
    NH = B * H
    a3 = a.reshape(NH, S, S)
    b3 = b.reshape(NH, S, D)
    n_blocks = S // _BLK

    out = pl.pallas_call(
        _diag_matmul_kernel,
        grid=(n_blocks,),
        in_specs=[
            pl.BlockSpec((NH, _BLK, _BLK), lambda i: (0, i, i)),
            pl.BlockSpec((NH, _BLK, D), lambda i: (0, i, 0)),
        ],
        out_specs=pl.BlockSpec((NH, _BLK, D), lambda i: (0, i, 0)),
        out_shape=jax.ShapeDtypeStruct((NH, S, D), jnp.float32),
    )(a3, b3)

    return out.reshape(B, H, S, D)


# speedup vs baseline: 1.9095x; 1.0034x over previous
import jax
import jax.numpy as jnp
from jax.experimental import pallas as pl
from jax.experimental.pallas import tpu as pltpu


_BLK = 256


def _diag_matmul_kernel(a_ref, b_ref, out_ref):
    out_ref[...] = jax.lax.dot_general(
        a_ref[...], b_ref[...],
        dimension_numbers=(((2,), (1,)), ((0,), (0,))),
        preferred_element_type=jnp.float32,
    )


def kernel(a, b):
    B, H, S, _ = a.shape
    D = b.shape[-1]
    NH = B * H
    a3 = a.reshape(NH, S, S)
    b3 = b.reshape(NH, S, D)
    n_blocks = S // _BLK

    out = pl.pallas_call(
        _diag_matmul_kernel,
        grid=(n_blocks,),
        in_specs=[
            pl.BlockSpec((NH, _BLK, _BLK), lambda i: (0, i, i)),
            pl.BlockSpec((NH, _BLK, D), lambda i: (0, i, 0)),
        ],
        out_specs=pl.BlockSpec((NH, _BLK, D), lambda i: (0, i, 0)),
        out_shape=jax.ShapeDtypeStruct((NH, S, D), jnp.float32),
        compiler_params=pltpu.CompilerParams(
            dimension_semantics=("parallel",),
        ),
    )(a3, b3)

    return out.reshape(B, H, S, D)


# speedup vs baseline: 6.4011x; 3.3523x over previous
import jax
import jax.numpy as jnp
from jax.experimental import pallas as pl
from jax.experimental.pallas import tpu as pltpu


_BLK = 256


def _diag_matmul_kernel(bt_ref, a_ref, out_ref):
    out_ref[...] = jax.lax.dot_general(
        bt_ref[...], a_ref[...],
        dimension_numbers=(((2,), (2,)), ((0,), (0,))),
        preferred_element_type=jnp.float32,
    )


def kernel(a, b):
    B, H, S, _ = a.shape
    D = b.shape[-1]
    NH = B * H
    a3 = a.reshape(NH, S, S)
    bt = jnp.swapaxes(b.reshape(NH, S, D), 1, 2)
    n_blocks = S // _BLK

    out_t = pl.pallas_call(
        _diag_matmul_kernel,
        grid=(n_blocks,),
        in_specs=[
            pl.BlockSpec((NH, D, _BLK), lambda i: (0, 0, i)),
            pl.BlockSpec((NH, _BLK, _BLK), lambda i: (0, i, i)),
        ],
        out_specs=pl.BlockSpec((NH, D, _BLK), lambda i: (0, 0, i)),
        out_shape=jax.ShapeDtypeStruct((NH, D, S), jnp.float32),
        compiler_params=pltpu.CompilerParams(
            dimension_semantics=("parallel",),
        ),
    )(bt, a3)

    return jnp.swapaxes(out_t, 1, 2).reshape(B, H, S, D)
